# baseline (speedup 1.0000x reference)
import functools

import jax
import jax.numpy as jnp
from jax import lax
from jax.experimental import pallas as pl
from jax.experimental.pallas import tpu as pltpu
from jax.experimental.pallas import tpu_sc as plsc


def _make_kernel(B, L, V, D, NW, b_per_w, n_ch, CH, CH_PAD):
    NC = 2
    NS = 16
    mesh = plsc.VectorSubcoreMesh(
        core_axis_name="c", subcore_axis_name="s", num_cores=NC, num_subcores=NS
    )

    @functools.partial(
        pl.kernel,
        mesh=mesh,
        out_type=jax.ShapeDtypeStruct((B, D), jnp.float32),
        compiler_params=pltpu.CompilerParams(use_tc_tiling_on_sc=False),
        scratch_types=[
            pltpu.VMEM((n_ch, CH_PAD), jnp.int32),
            pltpu.VMEM((2, CH_PAD, D), jnp.float32),
            pltpu.VMEM((b_per_w, D), jnp.float32),
            pltpu.SemaphoreType.DMA,
            pltpu.SemaphoreType.DMA,
        ],
    )
    def k(ids_hbm, table_hbm, out_hbm, idx_v, buf_v, out_v, sem0, sem1):
        wid = lax.axis_index("s") * NC + lax.axis_index("c")
        inv_l = jnp.float32(1.0 / L)
        nvec = D // 16

        pltpu.sync_copy(ids_hbm.at[wid], idx_v)

        def start(j, slot, sem):
            pltpu.async_copy(table_hbm.at[idx_v.at[j]], buf_v.at[slot], sem)

        def drain(slot, sem):
            pltpu.make_async_copy(table_hbm.at[idx_v.at[0]], buf_v.at[slot], sem).wait()

        def reduce_chunk(slot, acc):
            def body(r, a):
                return tuple(
                    a[q] + buf_v[slot, r, pl.ds(q * 16, 16)] for q in range(nvec)
                )
            return lax.fori_loop(0, CH, body, acc, unroll=2)

        start(0, 0, sem0)
        start(1, 1, sem1)

        def outer(b, carry):
            del carry
            zeros = tuple(jnp.zeros((16,), jnp.float32) for _ in range(nvec))
            drain(0, sem0)
            acc = reduce_chunk(0, zeros)

            @pl.when(b < b_per_w - 1)
            def _():
                start(2 * b + 2, 0, sem0)

            drain(1, sem1)
            acc = reduce_chunk(1, acc)

            @pl.when(b < b_per_w - 1)
            def _():
                start(2 * b + 3, 1, sem1)

            for q in range(nvec):
                out_v[b, pl.ds(q * 16, 16)] = acc[q] * inv_l
            return 0

        lax.fori_loop(0, b_per_w, outer, 0)

        pltpu.sync_copy(out_v, out_hbm.at[pl.ds(wid * b_per_w, b_per_w)])

    return k


def kernel(input_ids, pretrained_embeddings):
    B, L = input_ids.shape
    V, D = pretrained_embeddings.shape
    NW = 32
    b_per_w = B // NW
    CH = L // 2
    n_ch = b_per_w * 2
    CH_PAD = ((CH + 7) // 8) * 8

    ids = input_ids.reshape(NW, n_ch, CH)
    if CH_PAD != CH:
        pad = jnp.zeros((NW, n_ch, CH_PAD - CH), jnp.int32)
        ids = jnp.concatenate([ids, pad], axis=-1)

    k = _make_kernel(B, L, V, D, NW, b_per_w, n_ch, CH, CH_PAD)
    return k(ids, pretrained_embeddings)


# speedup vs baseline: 1.9392x; 1.9392x over previous
import functools

import jax
import jax.numpy as jnp
from jax import lax
from jax.experimental import pallas as pl
from jax.experimental.pallas import tpu as pltpu
from jax.experimental.pallas import tpu_sc as plsc

_NBUF = 4


def _make_kernel(B, L, V, D, NW, b_per_w):
    NC = 2
    NS = 16
    mesh = plsc.VectorSubcoreMesh(
        core_axis_name="c", subcore_axis_name="s", num_cores=NC, num_subcores=NS
    )
    nvec = D // 16
    n_grp = b_per_w // _NBUF

    @functools.partial(
        pl.kernel,
        mesh=mesh,
        out_type=jax.ShapeDtypeStruct((B, D), jnp.float32),
        compiler_params=pltpu.CompilerParams(use_tc_tiling_on_sc=False),
        scratch_types=[
            pltpu.VMEM((b_per_w, L), jnp.int32),
            pltpu.VMEM((_NBUF, L, D), jnp.float32),
            pltpu.VMEM((b_per_w, D), jnp.float32),
            [pltpu.SemaphoreType.DMA] * _NBUF,
        ],
    )
    def k(ids_hbm, table_hbm, out_hbm, idx_v, buf_v, out_v, sems):
        wid = lax.axis_index("s") * NC + lax.axis_index("c")
        inv_l = jnp.float32(1.0 / L)

        pltpu.sync_copy(ids_hbm.at[pl.ds(wid * b_per_w, b_per_w)], idx_v)

        def start(b, slot):
            pltpu.async_copy(table_hbm.at[idx_v.at[b]], buf_v.at[slot], sems[slot])

        def drain(slot):
            pltpu.make_async_copy(
                table_hbm.at[idx_v.at[0]], buf_v.at[slot], sems[slot]
            ).wait()

        def reduce_buf(slot, b):
            zeros = tuple(jnp.zeros((16,), jnp.float32) for _ in range(2 * nvec))

            def body(r, a):
                out = []
                for q in range(nvec):
                    out.append(a[q] + buf_v[slot, 2 * r, pl.ds(q * 16, 16)])
                for q in range(nvec):
                    out.append(a[nvec + q] + buf_v[slot, 2 * r + 1, pl.ds(q * 16, 16)])
                return tuple(out)

            acc = lax.fori_loop(0, L // 2, body, zeros, unroll=2)
            for q in range(nvec):
                out_v[b, pl.ds(q * 16, 16)] = (acc[q] + acc[nvec + q]) * inv_l

        for i in range(_NBUF):
            start(i, i)

        def outer(g, carry):
            del carry
            for i in range(_NBUF):
                b = g * _NBUF + i
                drain(i)
                reduce_buf(i, b)

                @pl.when(b < b_per_w - _NBUF)
                def _():
                    start(b + _NBUF, i)

            return 0

        lax.fori_loop(0, n_grp, outer, 0)

        pltpu.sync_copy(out_v, out_hbm.at[pl.ds(wid * b_per_w, b_per_w)])

    return k


def kernel(input_ids, pretrained_embeddings):
    B, L = input_ids.shape
    V, D = pretrained_embeddings.shape
    NW = 32
    b_per_w = B // NW
    k = _make_kernel(B, L, V, D, NW, b_per_w)
    return k(input_ids, pretrained_embeddings)
